# baseline (speedup 1.0000x reference)
import functools

import jax
import jax.numpy as jnp
from jax import lax
from jax.experimental import pallas as pl
from jax.experimental.pallas import tpu as pltpu
from jax.experimental.pallas import tpu_sc as plsc

K = 8192
D = 256
B = 16384
COMMIT = 0.25

RB = 128


def _argmin_body(wrow_ref, wall_ref, a_ref, wsum_ref, hn2_ref, hl_ref):
    @pl.when(pl.program_id(0) == 0)
    def _():
        wb0 = wall_ref[...]
        ones = jnp.ones((1, D), jnp.bfloat16)
        n2 = lax.dot_general(
            ones, wb0 * wb0, (((1,), (1,)), ((), ())),
            preferred_element_type=jnp.float32)
        wsum_ref[0, 0] = jnp.sum(n2)
        hn2_ref[...] = 0.5 * n2
        ki = lax.broadcasted_iota(jnp.int32, (K, 2), 0)
        col = lax.broadcasted_iota(jnp.int32, (K, 2), 1)
        hl_ref[...] = jnp.where(col == 0, ki // 64, ki % 64).astype(jnp.bfloat16)

    s = lax.dot_general(wrow_ref[...], wall_ref[...], (((1,), (1,)), ((), ())),
                        preferred_element_type=jnp.float32)
    score = s - hn2_ref[...]
    m = jnp.max(score, axis=1, keepdims=True)
    match = (score >= m).astype(jnp.bfloat16)
    hilo = lax.dot_general(match, hl_ref[...], (((1,), (0,)), ((), ())),
                           preferred_element_type=jnp.float32)
    a_ref[...] = (hilo[:, 0:1] * 64.0 + hilo[:, 1:2]).astype(jnp.int32)


def _codebook_argmin(wb):
    return pl.pallas_call(
        _argmin_body,
        grid=(K // RB,),
        in_specs=[
            pl.BlockSpec((RB, D), lambda i: (i, 0)),
            pl.BlockSpec((K, D), lambda i: (0, 0)),
        ],
        out_specs=[
            pl.BlockSpec((RB, 1), lambda i: (i, 0)),
            pl.BlockSpec(memory_space=pltpu.SMEM),
        ],
        out_shape=[
            jax.ShapeDtypeStruct((K, 1), jnp.int32),
            jax.ShapeDtypeStruct((1, 1), jnp.float32),
        ],
        scratch_shapes=[
            pltpu.VMEM((1, K), jnp.float32),
            pltpu.VMEM((K, 2), jnp.bfloat16),
        ],
    )(wb, wb)


_NW = 32
_BPW = B // _NW
_NCH = 4
_CH = _BPW // _NCH
_KPW = K // _NW
_KCH = _KPW // _CH


def _wq_body(a_hbm, w_hbm, wq_hbm, idx_v, rows_v, rows2_v, gs0, gs1, ws0, ws1):
    wid = lax.axis_index("s") * 2 + lax.axis_index("c")
    base = wid * _KPW
    bufs, gsem, wsem = (rows_v, rows2_v), (gs0, gs1), (ws0, ws1)
    for j in range(_KCH):
        pltpu.sync_copy(a_hbm.at[pl.ds(base + j * _CH, _CH)], idx_v.at[j])
    g = [pltpu.async_copy(w_hbm.at[idx_v.at[j]], bufs[j], gsem[j])
         for j in range(_KCH)]
    w = []
    for j in range(_KCH):
        g[j].wait()
        w.append(pltpu.async_copy(
            bufs[j], wq_hbm.at[pl.ds(base + j * _CH, _CH)], wsem[j]))
    for c in w:
        c.wait()


def _emb_body(x_hbm, w_hbm, out_hbm, xidx_v, rows_v, rows2_v, gs0, gs1, ws0, ws1):
    wid = lax.axis_index("s") * 2 + lax.axis_index("c")
    base = wid * _BPW
    bufs, gsem, wsem = (rows_v, rows2_v), (gs0, gs1), (ws0, ws1)
    for j in range(_NCH):
        pltpu.sync_copy(x_hbm.at[pl.ds(base + j * _CH, _CH)], xidx_v.at[j])
    g = [pltpu.async_copy(w_hbm.at[xidx_v.at[j]], bufs[j], gsem[j])
         for j in range(2)]
    w = []
    for j in range(_NCH):
        b = j % 2
        g[j].wait()
        w.append(pltpu.async_copy(
            bufs[b], out_hbm.at[pl.ds(base + j * _CH, _CH)], wsem[b]))
        if j + 2 < _NCH:
            w[j].wait()
            g.append(pltpu.async_copy(
                w_hbm.at[xidx_v.at[j + 2]], bufs[b], gsem[b]))
    w[_NCH - 2].wait()
    w[_NCH - 1].wait()


@functools.cache
def _wq_gather():
    return pl.kernel(
        _wq_body,
        out_type=jax.ShapeDtypeStruct((K, D), jnp.float32),
        mesh=plsc.VectorSubcoreMesh(core_axis_name="c", subcore_axis_name="s"),
        scratch_types=[
            pltpu.VMEM((_KCH, _CH), jnp.int32),
            pltpu.VMEM((_CH, D), jnp.float32),
            pltpu.VMEM((_CH, D), jnp.float32),
            pltpu.SemaphoreType.DMA,
            pltpu.SemaphoreType.DMA,
            pltpu.SemaphoreType.DMA,
            pltpu.SemaphoreType.DMA,
        ],
    )


@functools.cache
def _row_gather():
    return pl.kernel(
        _emb_body,
        out_type=jax.ShapeDtypeStruct((B, D), jnp.float32),
        mesh=plsc.VectorSubcoreMesh(core_axis_name="c", subcore_axis_name="s"),
        scratch_types=[
            pltpu.VMEM((_NCH, _CH), jnp.int32),
            pltpu.VMEM((_CH, D), jnp.float32),
            pltpu.VMEM((_CH, D), jnp.float32),
            pltpu.SemaphoreType.DMA,
            pltpu.SemaphoreType.DMA,
            pltpu.SemaphoreType.DMA,
            pltpu.SemaphoreType.DMA,
        ],
    )


DB = 512


def _diff_body(wsum_ref, xe_ref, q_ref, diff_ref, loss_ref, acc_ref):
    i = pl.program_id(0)
    d = q_ref[...] - xe_ref[...]
    diff_ref[...] = d
    ps = jnp.sum(d * d)
    acc_ref[0] = jnp.where(i == 0, ps, acc_ref[0] + ps)

    @pl.when(i == pl.num_programs(0) - 1)
    def _():
        loss_ref[0, 0] = acc_ref[0] / B + COMMIT * wsum_ref[0, 0]


def _diff_loss(wsum, xe, q):
    return pl.pallas_call(
        _diff_body,
        grid=(B // DB,),
        in_specs=[
            pl.BlockSpec(memory_space=pltpu.SMEM),
            pl.BlockSpec((DB, D), lambda i: (i, 0)),
            pl.BlockSpec((DB, D), lambda i: (i, 0)),
        ],
        out_specs=[
            pl.BlockSpec((DB, D), lambda i: (i, 0)),
            pl.BlockSpec(memory_space=pltpu.SMEM),
        ],
        out_shape=[
            jax.ShapeDtypeStruct((B, D), jnp.float32),
            jax.ShapeDtypeStruct((1, 1), jnp.float32),
        ],
        scratch_shapes=[pltpu.SMEM((1,), jnp.float32)],
    )(wsum, xe, q)


def kernel(x, W):
    xi = x.astype(jnp.int32)
    xe = _row_gather()(xi, W)
    wb = W.astype(jnp.bfloat16)
    a, wsum = _codebook_argmin(wb)
    wq = _wq_gather()(a.reshape(K), W)
    q = _row_gather()(xi, wq)
    diff, loss = _diff_loss(wsum, xe, q)
    return (loss[0, 0], q, diff)


# speedup vs baseline: 1.3224x; 1.3224x over previous
import functools

import jax
import jax.numpy as jnp
from jax import lax
from jax.experimental import pallas as pl
from jax.experimental.pallas import tpu as pltpu
from jax.experimental.pallas import tpu_sc as plsc

K = 8192
D = 256
B = 16384
COMMIT = 0.25

RB = 256


def _argmin_body(wrow_ref, wall_ref, a_ref, wsum_ref, hn2_ref, hl_ref):
    @pl.when(pl.program_id(0) == 0)
    def _():
        wb0 = wall_ref[...]
        ones = jnp.ones((1, D), jnp.bfloat16)
        n2 = lax.dot_general(
            ones, wb0 * wb0, (((1,), (1,)), ((), ())),
            preferred_element_type=jnp.float32)
        wsum_ref[0, 0] = jnp.sum(n2)
        hn2_ref[...] = 0.5 * n2
        ki = lax.broadcasted_iota(jnp.int32, (K, 2), 0)
        col = lax.broadcasted_iota(jnp.int32, (K, 2), 1)
        hl_ref[...] = jnp.where(col == 0, ki // 64, ki % 64).astype(jnp.bfloat16)

    s = lax.dot_general(wrow_ref[...], wall_ref[...], (((1,), (1,)), ((), ())),
                        preferred_element_type=jnp.float32)
    score = s - hn2_ref[...]
    m = jnp.max(score, axis=1, keepdims=True)
    match = (score >= m).astype(jnp.bfloat16)
    hilo = lax.dot_general(match, hl_ref[...], (((1,), (0,)), ((), ())),
                           preferred_element_type=jnp.float32)
    a_ref[...] = (hilo[:, 0:1] * 64.0 + hilo[:, 1:2]).astype(jnp.int32)


def _codebook_argmin(wb):
    return pl.pallas_call(
        _argmin_body,
        grid=(K // RB,),
        in_specs=[
            pl.BlockSpec((RB, D), lambda i: (i, 0)),
            pl.BlockSpec((K, D), lambda i: (0, 0)),
        ],
        out_specs=[
            pl.BlockSpec((RB, 1), lambda i: (i, 0)),
            pl.BlockSpec(memory_space=pltpu.SMEM),
        ],
        out_shape=[
            jax.ShapeDtypeStruct((K, 1), jnp.int32),
            jax.ShapeDtypeStruct((1, 1), jnp.float32),
        ],
        scratch_shapes=[
            pltpu.VMEM((1, K), jnp.float32),
            pltpu.VMEM((K, 2), jnp.bfloat16),
        ],
    )(wb, wb)


_NW = 32
_BPW = B // _NW
_NCH = 4
_CH = _BPW // _NCH
_KPW = K // _NW
_KCH = _KPW // _CH


def _wq_body(a_hbm, w_hbm, wq_hbm, idx_v, rows_v, rows2_v, gs0, gs1, ws0, ws1):
    wid = lax.axis_index("s") * 2 + lax.axis_index("c")
    base = wid * _KPW
    bufs, gsem, wsem = (rows_v, rows2_v), (gs0, gs1), (ws0, ws1)
    for j in range(_KCH):
        pltpu.sync_copy(a_hbm.at[pl.ds(base + j * _CH, _CH)], idx_v.at[j])
    g = [pltpu.async_copy(w_hbm.at[idx_v.at[j]], bufs[j], gsem[j])
         for j in range(_KCH)]
    w = []
    for j in range(_KCH):
        g[j].wait()
        w.append(pltpu.async_copy(
            bufs[j], wq_hbm.at[pl.ds(base + j * _CH, _CH)], wsem[j]))
    for c in w:
        c.wait()


def _emb_body(x_hbm, w_hbm, out_hbm, xidx_v, rows_v, rows2_v, gs0, gs1, ws0, ws1):
    wid = lax.axis_index("s") * 2 + lax.axis_index("c")
    base = wid * _BPW
    bufs, gsem, wsem = (rows_v, rows2_v), (gs0, gs1), (ws0, ws1)
    for j in range(_NCH):
        pltpu.sync_copy(x_hbm.at[pl.ds(base + j * _CH, _CH)], xidx_v.at[j])
    g = [pltpu.async_copy(w_hbm.at[xidx_v.at[j]], bufs[j], gsem[j])
         for j in range(2)]
    w = []
    for j in range(_NCH):
        b = j % 2
        g[j].wait()
        w.append(pltpu.async_copy(
            bufs[b], out_hbm.at[pl.ds(base + j * _CH, _CH)], wsem[b]))
        if j + 2 < _NCH:
            w[j].wait()
            g.append(pltpu.async_copy(
                w_hbm.at[xidx_v.at[j + 2]], bufs[b], gsem[b]))
    w[_NCH - 2].wait()
    w[_NCH - 1].wait()


@functools.cache
def _wq_gather():
    return pl.kernel(
        _wq_body,
        out_type=jax.ShapeDtypeStruct((K, D), jnp.float32),
        mesh=plsc.VectorSubcoreMesh(core_axis_name="c", subcore_axis_name="s"),
        scratch_types=[
            pltpu.VMEM((_KCH, _CH), jnp.int32),
            pltpu.VMEM((_CH, D), jnp.float32),
            pltpu.VMEM((_CH, D), jnp.float32),
            pltpu.SemaphoreType.DMA,
            pltpu.SemaphoreType.DMA,
            pltpu.SemaphoreType.DMA,
            pltpu.SemaphoreType.DMA,
        ],
    )


@functools.cache
def _row_gather():
    return pl.kernel(
        _emb_body,
        out_type=jax.ShapeDtypeStruct((B, D), jnp.float32),
        mesh=plsc.VectorSubcoreMesh(core_axis_name="c", subcore_axis_name="s"),
        scratch_types=[
            pltpu.VMEM((_NCH, _CH), jnp.int32),
            pltpu.VMEM((_CH, D), jnp.float32),
            pltpu.VMEM((_CH, D), jnp.float32),
            pltpu.SemaphoreType.DMA,
            pltpu.SemaphoreType.DMA,
            pltpu.SemaphoreType.DMA,
            pltpu.SemaphoreType.DMA,
        ],
    )


DB = 512


def _diff_body(wsum_ref, xe_ref, q_ref, diff_ref, loss_ref, acc_ref):
    i = pl.program_id(0)
    d = q_ref[...] - xe_ref[...]
    diff_ref[...] = d
    ps = jnp.sum(d * d)
    acc_ref[0] = jnp.where(i == 0, ps, acc_ref[0] + ps)

    @pl.when(i == pl.num_programs(0) - 1)
    def _():
        loss_ref[0, 0] = acc_ref[0] / B + COMMIT * wsum_ref[0, 0]


def _diff_loss(wsum, xe, q):
    return pl.pallas_call(
        _diff_body,
        grid=(B // DB,),
        in_specs=[
            pl.BlockSpec(memory_space=pltpu.SMEM),
            pl.BlockSpec((DB, D), lambda i: (i, 0)),
            pl.BlockSpec((DB, D), lambda i: (i, 0)),
        ],
        out_specs=[
            pl.BlockSpec((DB, D), lambda i: (i, 0)),
            pl.BlockSpec(memory_space=pltpu.SMEM),
        ],
        out_shape=[
            jax.ShapeDtypeStruct((B, D), jnp.float32),
            jax.ShapeDtypeStruct((1, 1), jnp.float32),
        ],
        scratch_shapes=[pltpu.SMEM((1,), jnp.float32)],
    )(wsum, xe, q)


def kernel(x, W):
    xi = x.astype(jnp.int32)
    xe = _row_gather()(xi, W)
    wb = W.astype(jnp.bfloat16)
    a, wsum = _codebook_argmin(wb)
    wq = _wq_gather()(a.reshape(K), W)
    q = _row_gather()(xi, wq)
    diff, loss = _diff_loss(wsum, xe, q)
    return (loss[0, 0], q, diff)
